# Initial kernel scaffold; baseline (speedup 1.0000x reference)
import jax
import jax.numpy as jnp
from jax.experimental import pallas as pl


def kernel(params, atom_list, bond_list, atom_degree_list, bond_degree_list, atom_mask, xyz_feature):
    raise NotImplementedError("write your pallas kernel here")


# speedup vs baseline: 7.6824x; 7.6824x over previous
import jax
import jax.numpy as jnp
from jax.experimental import pallas as pl

B, L, NBR = 256, 64, 6
AF, BF, FP, NC = 39, 10, 128, 2
MB = 32

_HI = jax.lax.Precision.HIGHEST


def _mm(a, b):
    return jax.lax.dot_general(a, b, (((1,), (0,)), ((), ())),
                               precision=_HI, preferred_element_type=jnp.float32)


def _bmm(a, b):
    return jax.lax.dot_general(a, b, (((2,), (1,)), ((0,), (0,))),
                               precision=_HI, preferred_element_type=jnp.float32)


def _leaky(x):
    return jnp.where(x >= 0, x, 0.01 * x)


def _elu(x):
    return jnp.where(x > 0, x, jnp.exp(jnp.minimum(x, 0.0)) - 1.0)


def _sig(x):
    return 1.0 / (1.0 + jnp.exp(-x))


def _gru(x2d, h2d, wihT, bih, whhT, bhh):
    gi = _mm(x2d, wihT) + bih
    gh = _mm(h2d, whhT) + bhh
    r = _sig(gi[:, :FP] + gh[:, :FP])
    z = _sig(gi[:, FP:2 * FP] + gh[:, FP:2 * FP])
    n = jnp.tanh(gi[:, 2 * FP:] + r * gh[:, 2 * FP:])
    return (1.0 - z) * n + z * h2d


def _gnn_body(atom_ref, bond_ref, ideg_ref, bdeg_ref, mask_ref,
              wfT_ref, bf_ref, waT_ref, wbT_ref, bnb_ref,
              wa1_0_ref, wa2_0_ref, bal0_ref, wattT0_ref, batt0_ref,
              wihT0_ref, bih0_ref, whhT0_ref, bhh0_ref,
              wa1_1_ref, wa2_1_ref, bal1_ref, wattT1_ref, batt1_ref,
              wihT1_ref, bih1_ref, whhT1_ref, bhh1_ref,
              wm1_ref, wm2_ref, bm_ref, wmaT_ref, bma_ref,
              wihTm_ref, bihm_ref, whhTm_ref, bhhm_ref,
              lng_ref, lnb_ref, out_ref):
    atom2 = atom_ref[:].reshape(MB * L, AF)
    bond2 = bond_ref[:].reshape(MB * L, BF)
    mask = mask_ref[:]
    idx_a = ideg_ref[:]
    idx_b = bdeg_ref[:]

    af2d = _leaky(_mm(atom2, wfT_ref[:]) + bf_ref[:])
    A3 = _mm(atom2, waT_ref[:]).reshape(MB, L, FP)
    B3 = _mm(bond2, wbT_ref[:]).reshape(MB, L, FP)
    bnb = bnb_ref[:][None]
    af3 = af2d.reshape(MB, L, FP)

    iota = jax.lax.broadcasted_iota(jnp.int32, (MB, L, L), 2)

    oh_a, N0, amask, smask = [], [], [], []
    for n in range(NBR):
        ia = idx_a[:, :, n]
        ib = idx_b[:, :, n]
        oa = (ia[:, :, None] == iota).astype(jnp.float32)
        ob = (ib[:, :, None] == iota).astype(jnp.float32)
        oh_a.append(oa)
        pad = ia == (L - 1)
        amask.append(jnp.where(pad, 0.0, 1.0))
        smask.append(jnp.where(pad, -9.0e8, 0.0))
        N0.append(_leaky(_bmm(oa, A3) + _bmm(ob, B3) + bnb))

    def attn(s1, s2_list, bal, feats):
        sc = [_leaky(s1 + s2_list[n] + bal) + smask[n] for n in range(NBR)]
        mx = sc[0]
        for n in range(1, NBR):
            mx = jnp.maximum(mx, sc[n])
        e = [jnp.exp(sc[n] - mx) for n in range(NBR)]
        zden = e[0]
        for n in range(1, NBR):
            zden = zden + e[n]
        w = [e[n] / zden * amask[n] for n in range(NBR)]
        wsum = w[0]
        for n in range(1, NBR):
            wsum = wsum + w[n]
        agg = w[0][:, :, None] * feats[0]
        for n in range(1, NBR):
            agg = agg + w[n][:, :, None] * feats[n]
        return agg, wsum, w

    wa1_0 = wa1_0_ref[:][None]
    wa2_0 = wa2_0_ref[:][None]
    s1_0 = jnp.sum(af3 * wa1_0, axis=-1)
    s2_0 = [jnp.sum(N0[n] * wa2_0, axis=-1) for n in range(NBR)]
    agg0, wsum0, _ = attn(s1_0, s2_0, bal0_ref[:], N0)
    ctx0 = _elu(_mm(agg0.reshape(MB * L, FP), wattT0_ref[:]).reshape(MB, L, FP)
                + wsum0[:, :, None] * batt0_ref[:][None])
    h1 = _gru(ctx0.reshape(MB * L, FP), af2d,
              wihT0_ref[:], bih0_ref[:], whhT0_ref[:], bhh0_ref[:])

    act3 = jnp.maximum(h1, 0.0).reshape(MB, L, FP)
    wa1_1 = wa1_1_ref[:][None]
    wa2_1 = wa2_1_ref[:][None]
    s1_1 = jnp.sum(act3 * wa1_1, axis=-1)
    zsc = jnp.sum(act3 * wa2_1, axis=-1)
    s2_1 = [jnp.sum(oh_a[n] * zsc[:, None, :], axis=-1) for n in range(NBR)]
    sc1 = [_leaky(s1_1 + s2_1[n] + bal1_ref[:]) + smask[n] for n in range(NBR)]
    mx = sc1[0]
    for n in range(1, NBR):
        mx = jnp.maximum(mx, sc1[n])
    e1 = [jnp.exp(sc1[n] - mx) for n in range(NBR)]
    zden = e1[0]
    for n in range(1, NBR):
        zden = zden + e1[n]
    w1 = [e1[n] / zden * amask[n] for n in range(NBR)]
    wsum1 = w1[0]
    for n in range(1, NBR):
        wsum1 = wsum1 + w1[n]
    wadj = w1[0][:, :, None] * oh_a[0]
    for n in range(1, NBR):
        wadj = wadj + w1[n][:, :, None] * oh_a[n]
    agg1 = _bmm(wadj, act3)
    ctx1 = _elu(_mm(agg1.reshape(MB * L, FP), wattT1_ref[:]).reshape(MB, L, FP)
                + wsum1[:, :, None] * batt1_ref[:][None])
    h2 = _gru(ctx1.reshape(MB * L, FP), h1,
              wihT1_ref[:], bih1_ref[:], whhT1_ref[:], bhh1_ref[:])

    act5 = jnp.maximum(h2, 0.0).reshape(MB, L, FP)
    mol = jnp.sum(act5 * mask[:, :, None], axis=1)
    act_mol = jnp.maximum(mol, 0.0)
    mmask_add = jnp.where(mask == 0.0, -9.0e8, 0.0)
    wm1 = wm1_ref[:]
    wm2 = wm2_ref[:][None]
    for _t in range(2):
        s1m = jnp.sum(act_mol * wm1, axis=-1, keepdims=True)
        s2m = jnp.sum(act5 * wm2, axis=-1)
        sc = _leaky(s1m + s2m + bm_ref[:]) + mmask_add
        mx2 = jnp.max(sc, axis=1, keepdims=True)
        e2 = jnp.exp(sc - mx2)
        w2 = e2 / jnp.sum(e2, axis=1, keepdims=True) * mask
        wsum2 = jnp.sum(w2, axis=1, keepdims=True)
        aggm = jnp.sum(act5 * w2[:, :, None], axis=1)
        ctxm = _elu(_mm(aggm, wmaT_ref[:]) + wsum2 * bma_ref[:])
        mol_new = _gru(ctxm, mol, wihTm_ref[:], bihm_ref[:],
                       whhTm_ref[:], bhhm_ref[:])
        mu = jnp.mean(mol_new, axis=1, keepdims=True)
        var = jnp.mean((mol_new - mu) ** 2, axis=1, keepdims=True)
        mol = (mol_new - mu) * jax.lax.rsqrt(var + 1e-5) * lng_ref[:] \
            + lnb_ref[:] + mol_new
        act_mol = jnp.maximum(mol, 0.0)
    out_ref[:] = mol


def _xyz_body(xyz_ref, mol_ref,
              w1T_ref, g1_ref, be1_ref, w2T_ref, g2_ref, be2_ref,
              w3T_ref, g3_ref, be3_ref,
              fc1T_ref, bfc1_ref, fc2T_ref, bfc2_ref,
              fc3T_ref, bfc3_ref, fc4T_ref, bfc4_ref, out_ref):
    N = B * L
    x0 = xyz_ref[:]
    z1 = _mm(x0, w1T_ref[:])
    mu1 = jnp.mean(z1, axis=0, keepdims=True)
    v1 = jnp.mean(z1 * z1, axis=0, keepdims=True) - mu1 * mu1
    x1 = _leaky((z1 - mu1) * jax.lax.rsqrt(v1 + 1e-5) * g1_ref[:] + be1_ref[:])
    z2 = _mm(x1, w2T_ref[:])
    mu2 = jnp.mean(z2, axis=0, keepdims=True)
    v2 = jnp.mean(z2 * z2, axis=0, keepdims=True) - mu2 * mu2
    x2 = _leaky((z2 - mu2) * jax.lax.rsqrt(v2 + 1e-5) * g2_ref[:] + be2_ref[:])

    CH = 2048
    ssum = jnp.zeros((1, 1024), jnp.float32)
    ssq = jnp.zeros((1, 1024), jnp.float32)
    mxs, mns = [], []
    for c in range(N // CH):
        z3c = _mm(x2[c * CH:(c + 1) * CH], w3T_ref[:])
        ssum = ssum + jnp.sum(z3c, axis=0, keepdims=True)
        ssq = ssq + jnp.sum(z3c * z3c, axis=0, keepdims=True)
        z3r = z3c.reshape(CH // L, L, 1024)
        mxs.append(jnp.max(z3r, axis=1))
        mns.append(jnp.min(z3r, axis=1))
    maxb = jnp.concatenate(mxs, axis=0)
    minb = jnp.concatenate(mns, axis=0)
    mu3 = ssum / N
    v3 = ssq / N - mu3 * mu3
    scale = jax.lax.rsqrt(v3 + 1e-5) * g3_ref[:]
    f_mx = _leaky((maxb - mu3) * scale + be3_ref[:])
    f_mn = _leaky((minb - mu3) * scale + be3_ref[:])
    x3 = jnp.maximum(f_mx, f_mn)
    h = jnp.maximum(_mm(x3, fc1T_ref[:]) + bfc1_ref[:], 0.0)
    h = jnp.maximum(_mm(h, fc2T_ref[:]) + bfc2_ref[:], 0.0)
    cat = jnp.concatenate([mol_ref[:], h], axis=1)
    f3 = jnp.maximum(_mm(cat, fc3T_ref[:]) + bfc3_ref[:], 0.0)
    logits = _mm(f3, fc4T_ref[:]) + bfc4_ref[:]
    mxl = jnp.max(logits, axis=1, keepdims=True)
    el = jnp.exp(logits - mxl)
    out_ref[:] = el / jnp.sum(el, axis=1, keepdims=True)


def _row(v):
    return jnp.asarray(v, jnp.float32).reshape(1, -1)


def kernel(params, atom_list, bond_list, atom_degree_list, bond_degree_list,
           atom_mask, xyz_feature):
    p = params
    wfT = p['atom_fc'][0].T
    bf = _row(p['atom_fc'][1])
    waT = p['neighbor_fc'][0][:, :AF].T
    wbT = p['neighbor_fc'][0][:, AF:].T
    bnb = _row(p['neighbor_fc'][1])

    def align_parts(r):
        W, b = p['align'][r]
        return _row(W[0, :FP]), _row(W[0, FP:]), _row(b)

    wa1_0, wa2_0, bal0 = align_parts(0)
    wa1_1, wa2_1, bal1 = align_parts(1)
    wattT0, batt0 = p['attend'][0][0].T, _row(p['attend'][0][1])
    wattT1, batt1 = p['attend'][1][0].T, _row(p['attend'][1][1])

    def gru_parts(g):
        return (g['W_ih'].T, _row(g['b_ih']), g['W_hh'].T, _row(g['b_hh']))

    wihT0, bih0, whhT0, bhh0 = gru_parts(p['gru'][0])
    wihT1, bih1, whhT1, bhh1 = gru_parts(p['gru'][1])
    wihTm, bihm, whhTm, bhhm = gru_parts(p['mol_gru'])
    Wm, bm = p['mol_align']
    wm1, wm2, bmr = _row(Wm[0, :FP]), _row(Wm[0, FP:]), _row(bm)
    wmaT, bma = p['mol_attend'][0].T, _row(p['mol_attend'][1])
    lng, lnb = _row(p['ln'][0]), _row(p['ln'][1])

    full = lambda shape: pl.BlockSpec(shape, lambda i: (0,) * len(shape))
    weight_args = [wfT, bf, waT, wbT, bnb,
                   wa1_0, wa2_0, bal0, wattT0, batt0, wihT0, bih0, whhT0, bhh0,
                   wa1_1, wa2_1, bal1, wattT1, batt1, wihT1, bih1, whhT1, bhh1,
                   wm1, wm2, bmr, wmaT, bma, wihTm, bihm, whhTm, bhhm, lng, lnb]

    mol_feature = pl.pallas_call(
        _gnn_body,
        grid=(B // MB,),
        in_specs=[
            pl.BlockSpec((MB, L, AF), lambda i: (i, 0, 0)),
            pl.BlockSpec((MB, L, BF), lambda i: (i, 0, 0)),
            pl.BlockSpec((MB, L, NBR), lambda i: (i, 0, 0)),
            pl.BlockSpec((MB, L, NBR), lambda i: (i, 0, 0)),
            pl.BlockSpec((MB, L), lambda i: (i, 0)),
        ] + [full(w.shape) for w in weight_args],
        out_specs=pl.BlockSpec((MB, FP), lambda i: (i, 0)),
        out_shape=jax.ShapeDtypeStruct((B, FP), jnp.float32),
    )(atom_list, bond_list, atom_degree_list, bond_degree_list, atom_mask,
      *weight_args)

    xyz2 = xyz_feature.reshape(B * L, 6)
    head_args = [p['conv1'][0].T, _row(p['bn1'][0]), _row(p['bn1'][1]),
                 p['conv2'][0].T, _row(p['bn2'][0]), _row(p['bn2'][1]),
                 p['conv3'][0].T, _row(p['bn3'][0]), _row(p['bn3'][1]),
                 p['fc1'][0].T, _row(p['fc1'][1]),
                 p['fc2'][0].T, _row(p['fc2'][1]),
                 p['fc3'][0].T, _row(p['fc3'][1]),
                 p['fc4'][0].T, _row(p['fc4'][1])]

    out = pl.pallas_call(
        _xyz_body,
        out_shape=jax.ShapeDtypeStruct((B, NC), jnp.float32),
    )(xyz2, mol_feature, *head_args)
    return out


# speedup vs baseline: 8.1787x; 1.0646x over previous
import jax
import jax.numpy as jnp
from jax.experimental import pallas as pl

B, L, NBR = 256, 64, 6
AF, BF, FP, NC = 39, 10, 128, 2
MB = 32

_HI = jax.lax.Precision.HIGHEST


def _mm(a, b):
    return jax.lax.dot_general(a, b, (((1,), (0,)), ((), ())),
                               precision=_HI, preferred_element_type=jnp.float32)


def _bmm(a, b):
    return jax.lax.dot_general(a, b, (((2,), (1,)), ((0,), (0,))),
                               precision=_HI, preferred_element_type=jnp.float32)


def _leaky(x):
    return jnp.where(x >= 0, x, 0.01 * x)


def _elu(x):
    return jnp.where(x > 0, x, jnp.exp(jnp.minimum(x, 0.0)) - 1.0)


def _sig(x):
    return 1.0 / (1.0 + jnp.exp(-x))


def _gru(x2d, h2d, wihT, bih, whhT, bhh):
    gi = _mm(x2d, wihT) + bih
    gh = _mm(h2d, whhT) + bhh
    r = _sig(gi[:, :FP] + gh[:, :FP])
    z = _sig(gi[:, FP:2 * FP] + gh[:, FP:2 * FP])
    n = jnp.tanh(gi[:, 2 * FP:] + r * gh[:, 2 * FP:])
    return (1.0 - z) * n + z * h2d


def _gnn_body(atom_ref, bond_ref, ideg_ref, bdeg_ref, mask_ref,
              wfT_ref, bf_ref, waT_ref, wbT_ref, bnb_ref,
              wa1_0_ref, wa2_0_ref, bal0_ref, wattT0_ref, batt0_ref,
              wihT0_ref, bih0_ref, whhT0_ref, bhh0_ref,
              wa1_1_ref, wa2_1_ref, bal1_ref, wattT1_ref, batt1_ref,
              wihT1_ref, bih1_ref, whhT1_ref, bhh1_ref,
              wm1_ref, wm2_ref, bm_ref, wmaT_ref, bma_ref,
              wihTm_ref, bihm_ref, whhTm_ref, bhhm_ref,
              lng_ref, lnb_ref, out_ref):
    atom2 = atom_ref[:].reshape(MB * L, AF)
    bond2 = bond_ref[:].reshape(MB * L, BF)
    mask = mask_ref[:]
    idx_a = ideg_ref[:]
    idx_b = bdeg_ref[:]

    af2d = _leaky(_mm(atom2, wfT_ref[:]) + bf_ref[:])
    A3 = _mm(atom2, waT_ref[:]).reshape(MB, L, FP)
    B3 = _mm(bond2, wbT_ref[:]).reshape(MB, L, FP)
    bnb = bnb_ref[:][None]
    af3 = af2d.reshape(MB, L, FP)

    iota2 = jax.lax.broadcasted_iota(jnp.int32, (MB, L, 2 * L), 2)

    oh, amask, smask = [], [], []
    for n in range(NBR):
        ia = idx_a[:, :, n]
        ib = idx_b[:, :, n] + L
        oh.append(jnp.where((ia[:, :, None] == iota2) | (ib[:, :, None] == iota2),
                            1.0, 0.0))
        pad = ia == (L - 1)
        amask.append(jnp.where(pad, 0.0, 1.0))
        smask.append(jnp.where(pad, -9.0e8, 0.0))
    OH = jnp.concatenate(oh, axis=1)
    T0 = jnp.concatenate([A3, B3], axis=1)
    N0full = _leaky(_bmm(OH, T0) + bnb)
    N0 = [N0full[:, n * L:(n + 1) * L, :] for n in range(NBR)]
    oh_a = [oh[n][:, :, :L] for n in range(NBR)]

    def attn(s1, s2_list, bal, feats):
        sc = [_leaky(s1 + s2_list[n] + bal) + smask[n] for n in range(NBR)]
        mx = sc[0]
        for n in range(1, NBR):
            mx = jnp.maximum(mx, sc[n])
        e = [jnp.exp(sc[n] - mx) for n in range(NBR)]
        zden = e[0]
        for n in range(1, NBR):
            zden = zden + e[n]
        w = [e[n] / zden * amask[n] for n in range(NBR)]
        wsum = w[0]
        for n in range(1, NBR):
            wsum = wsum + w[n]
        agg = w[0][:, :, None] * feats[0]
        for n in range(1, NBR):
            agg = agg + w[n][:, :, None] * feats[n]
        return agg, wsum, w

    wa1_0 = wa1_0_ref[:][None]
    wa2_0 = wa2_0_ref[:][None]
    s1_0 = jnp.sum(af3 * wa1_0, axis=-1)
    s2_0 = [jnp.sum(N0[n] * wa2_0, axis=-1) for n in range(NBR)]
    agg0, wsum0, _ = attn(s1_0, s2_0, bal0_ref[:], N0)
    ctx0 = _elu(_mm(agg0.reshape(MB * L, FP), wattT0_ref[:]).reshape(MB, L, FP)
                + wsum0[:, :, None] * batt0_ref[:][None])
    h1 = _gru(ctx0.reshape(MB * L, FP), af2d,
              wihT0_ref[:], bih0_ref[:], whhT0_ref[:], bhh0_ref[:])

    act3 = jnp.maximum(h1, 0.0).reshape(MB, L, FP)
    wa1_1 = wa1_1_ref[:][None]
    wa2_1 = wa2_1_ref[:][None]
    s1_1 = jnp.sum(act3 * wa1_1, axis=-1)
    zsc = jnp.sum(act3 * wa2_1, axis=-1)
    s2_1 = [jnp.sum(oh_a[n] * zsc[:, None, :], axis=-1) for n in range(NBR)]
    sc1 = [_leaky(s1_1 + s2_1[n] + bal1_ref[:]) + smask[n] for n in range(NBR)]
    mx = sc1[0]
    for n in range(1, NBR):
        mx = jnp.maximum(mx, sc1[n])
    e1 = [jnp.exp(sc1[n] - mx) for n in range(NBR)]
    zden = e1[0]
    for n in range(1, NBR):
        zden = zden + e1[n]
    w1 = [e1[n] / zden * amask[n] for n in range(NBR)]
    wsum1 = w1[0]
    for n in range(1, NBR):
        wsum1 = wsum1 + w1[n]
    wadj = w1[0][:, :, None] * oh_a[0]
    for n in range(1, NBR):
        wadj = wadj + w1[n][:, :, None] * oh_a[n]
    agg1 = _bmm(wadj, act3)
    ctx1 = _elu(_mm(agg1.reshape(MB * L, FP), wattT1_ref[:]).reshape(MB, L, FP)
                + wsum1[:, :, None] * batt1_ref[:][None])
    h2 = _gru(ctx1.reshape(MB * L, FP), h1,
              wihT1_ref[:], bih1_ref[:], whhT1_ref[:], bhh1_ref[:])

    act5 = jnp.maximum(h2, 0.0).reshape(MB, L, FP)
    mol = jnp.sum(act5 * mask[:, :, None], axis=1)
    act_mol = jnp.maximum(mol, 0.0)
    mmask_add = jnp.where(mask == 0.0, -9.0e8, 0.0)
    wm1 = wm1_ref[:]
    wm2 = wm2_ref[:][None]
    for _t in range(2):
        s1m = jnp.sum(act_mol * wm1, axis=-1, keepdims=True)
        s2m = jnp.sum(act5 * wm2, axis=-1)
        sc = _leaky(s1m + s2m + bm_ref[:]) + mmask_add
        mx2 = jnp.max(sc, axis=1, keepdims=True)
        e2 = jnp.exp(sc - mx2)
        w2 = e2 / jnp.sum(e2, axis=1, keepdims=True) * mask
        wsum2 = jnp.sum(w2, axis=1, keepdims=True)
        aggm = jnp.sum(act5 * w2[:, :, None], axis=1)
        ctxm = _elu(_mm(aggm, wmaT_ref[:]) + wsum2 * bma_ref[:])
        mol_new = _gru(ctxm, mol, wihTm_ref[:], bihm_ref[:],
                       whhTm_ref[:], bhhm_ref[:])
        mu = jnp.mean(mol_new, axis=1, keepdims=True)
        var = jnp.mean((mol_new - mu) ** 2, axis=1, keepdims=True)
        mol = (mol_new - mu) * jax.lax.rsqrt(var + 1e-5) * lng_ref[:] \
            + lnb_ref[:] + mol_new
        act_mol = jnp.maximum(mol, 0.0)
    out_ref[:] = mol


def _xyz_body(xyz_ref, mol_ref,
              w1T_ref, g1_ref, be1_ref, w2T_ref, g2_ref, be2_ref,
              w3T_ref, g3_ref, be3_ref,
              fc1T_ref, bfc1_ref, fc2T_ref, bfc2_ref,
              fc3T_ref, bfc3_ref, fc4T_ref, bfc4_ref, out_ref):
    N = B * L
    x0 = xyz_ref[:]
    z1 = _mm(x0, w1T_ref[:])
    mu1 = jnp.mean(z1, axis=0, keepdims=True)
    v1 = jnp.mean(z1 * z1, axis=0, keepdims=True) - mu1 * mu1
    x1 = _leaky((z1 - mu1) * jax.lax.rsqrt(v1 + 1e-5) * g1_ref[:] + be1_ref[:])
    z2 = _mm(x1, w2T_ref[:])
    mu2 = jnp.mean(z2, axis=0, keepdims=True)
    v2 = jnp.mean(z2 * z2, axis=0, keepdims=True) - mu2 * mu2
    x2 = _leaky((z2 - mu2) * jax.lax.rsqrt(v2 + 1e-5) * g2_ref[:] + be2_ref[:])

    CH = 2048
    ssum = jnp.zeros((1, 1024), jnp.float32)
    ssq = jnp.zeros((1, 1024), jnp.float32)
    mxs, mns = [], []
    for c in range(N // CH):
        z3c = _mm(x2[c * CH:(c + 1) * CH], w3T_ref[:])
        ssum = ssum + jnp.sum(z3c, axis=0, keepdims=True)
        ssq = ssq + jnp.sum(z3c * z3c, axis=0, keepdims=True)
        z3r = z3c.reshape(CH // L, L, 1024)
        mxs.append(jnp.max(z3r, axis=1))
        mns.append(jnp.min(z3r, axis=1))
    maxb = jnp.concatenate(mxs, axis=0)
    minb = jnp.concatenate(mns, axis=0)
    mu3 = ssum / N
    v3 = ssq / N - mu3 * mu3
    scale = jax.lax.rsqrt(v3 + 1e-5) * g3_ref[:]
    f_mx = _leaky((maxb - mu3) * scale + be3_ref[:])
    f_mn = _leaky((minb - mu3) * scale + be3_ref[:])
    x3 = jnp.maximum(f_mx, f_mn)
    h = jnp.maximum(_mm(x3, fc1T_ref[:]) + bfc1_ref[:], 0.0)
    h = jnp.maximum(_mm(h, fc2T_ref[:]) + bfc2_ref[:], 0.0)
    cat = jnp.concatenate([mol_ref[:], h], axis=1)
    f3 = jnp.maximum(_mm(cat, fc3T_ref[:]) + bfc3_ref[:], 0.0)
    logits = _mm(f3, fc4T_ref[:]) + bfc4_ref[:]
    mxl = jnp.max(logits, axis=1, keepdims=True)
    el = jnp.exp(logits - mxl)
    out_ref[:] = el / jnp.sum(el, axis=1, keepdims=True)


def _row(v):
    return jnp.asarray(v, jnp.float32).reshape(1, -1)


def kernel(params, atom_list, bond_list, atom_degree_list, bond_degree_list,
           atom_mask, xyz_feature):
    p = params
    wfT = p['atom_fc'][0].T
    bf = _row(p['atom_fc'][1])
    waT = p['neighbor_fc'][0][:, :AF].T
    wbT = p['neighbor_fc'][0][:, AF:].T
    bnb = _row(p['neighbor_fc'][1])

    def align_parts(r):
        W, b = p['align'][r]
        return _row(W[0, :FP]), _row(W[0, FP:]), _row(b)

    wa1_0, wa2_0, bal0 = align_parts(0)
    wa1_1, wa2_1, bal1 = align_parts(1)
    wattT0, batt0 = p['attend'][0][0].T, _row(p['attend'][0][1])
    wattT1, batt1 = p['attend'][1][0].T, _row(p['attend'][1][1])

    def gru_parts(g):
        return (g['W_ih'].T, _row(g['b_ih']), g['W_hh'].T, _row(g['b_hh']))

    wihT0, bih0, whhT0, bhh0 = gru_parts(p['gru'][0])
    wihT1, bih1, whhT1, bhh1 = gru_parts(p['gru'][1])
    wihTm, bihm, whhTm, bhhm = gru_parts(p['mol_gru'])
    Wm, bm = p['mol_align']
    wm1, wm2, bmr = _row(Wm[0, :FP]), _row(Wm[0, FP:]), _row(bm)
    wmaT, bma = p['mol_attend'][0].T, _row(p['mol_attend'][1])
    lng, lnb = _row(p['ln'][0]), _row(p['ln'][1])

    full = lambda shape: pl.BlockSpec(shape, lambda i: (0,) * len(shape))
    weight_args = [wfT, bf, waT, wbT, bnb,
                   wa1_0, wa2_0, bal0, wattT0, batt0, wihT0, bih0, whhT0, bhh0,
                   wa1_1, wa2_1, bal1, wattT1, batt1, wihT1, bih1, whhT1, bhh1,
                   wm1, wm2, bmr, wmaT, bma, wihTm, bihm, whhTm, bhhm, lng, lnb]

    mol_feature = pl.pallas_call(
        _gnn_body,
        grid=(B // MB,),
        in_specs=[
            pl.BlockSpec((MB, L, AF), lambda i: (i, 0, 0)),
            pl.BlockSpec((MB, L, BF), lambda i: (i, 0, 0)),
            pl.BlockSpec((MB, L, NBR), lambda i: (i, 0, 0)),
            pl.BlockSpec((MB, L, NBR), lambda i: (i, 0, 0)),
            pl.BlockSpec((MB, L), lambda i: (i, 0)),
        ] + [full(w.shape) for w in weight_args],
        out_specs=pl.BlockSpec((MB, FP), lambda i: (i, 0)),
        out_shape=jax.ShapeDtypeStruct((B, FP), jnp.float32),
    )(atom_list, bond_list, atom_degree_list, bond_degree_list, atom_mask,
      *weight_args)

    xyz2 = xyz_feature.reshape(B * L, 6)
    head_args = [p['conv1'][0].T, _row(p['bn1'][0]), _row(p['bn1'][1]),
                 p['conv2'][0].T, _row(p['bn2'][0]), _row(p['bn2'][1]),
                 p['conv3'][0].T, _row(p['bn3'][0]), _row(p['bn3'][1]),
                 p['fc1'][0].T, _row(p['fc1'][1]),
                 p['fc2'][0].T, _row(p['fc2'][1]),
                 p['fc3'][0].T, _row(p['fc3'][1]),
                 p['fc4'][0].T, _row(p['fc4'][1])]

    out = pl.pallas_call(
        _xyz_body,
        out_shape=jax.ShapeDtypeStruct((B, NC), jnp.float32),
    )(xyz2, mol_feature, *head_args)
    return out


# speedup vs baseline: 9.0540x; 1.1070x over previous
import jax
import jax.numpy as jnp
from jax.experimental import pallas as pl

B, L, NBR = 256, 64, 6
AF, BF, FP, NC = 39, 10, 128, 2
MB = 16

_HI = jax.lax.Precision.HIGHEST


def _mm(a, b):
    return jax.lax.dot_general(a, b, (((1,), (0,)), ((), ())),
                               precision=_HI, preferred_element_type=jnp.float32)


def _bmm(a, b):
    return jax.lax.dot_general(a, b, (((2,), (1,)), ((0,), (0,))),
                               precision=_HI, preferred_element_type=jnp.float32)


def _leaky(x):
    return jnp.where(x >= 0, x, 0.01 * x)


def _elu(x):
    return jnp.where(x > 0, x, jnp.exp(jnp.minimum(x, 0.0)) - 1.0)


def _sig(x):
    return 1.0 / (1.0 + jnp.exp(-x))


def _gru(x2d, h2d, wihT, bih, whhT, bhh):
    gi = _mm(x2d, wihT) + bih
    gh = _mm(h2d, whhT) + bhh
    r = _sig(gi[:, :FP] + gh[:, :FP])
    z = _sig(gi[:, FP:2 * FP] + gh[:, FP:2 * FP])
    n = jnp.tanh(gi[:, 2 * FP:] + r * gh[:, 2 * FP:])
    return (1.0 - z) * n + z * h2d


def _gnn_body(atom_ref, bond_ref, ideg_ref, bdeg_ref, mask_ref,
              wfT_ref, bf_ref, waT_ref, wbT_ref, bnb_ref,
              wa1_0_ref, wa2_0_ref, bal0_ref, wattT0_ref, batt0_ref,
              wihT0_ref, bih0_ref, whhT0_ref, bhh0_ref,
              wa1_1_ref, wa2_1_ref, bal1_ref, wattT1_ref, batt1_ref,
              wihT1_ref, bih1_ref, whhT1_ref, bhh1_ref,
              wm1_ref, wm2_ref, bm_ref, wmaT_ref, bma_ref,
              wihTm_ref, bihm_ref, whhTm_ref, bhhm_ref,
              lng_ref, lnb_ref, out_ref):
    atom2 = atom_ref[:].reshape(MB * L, AF)
    bond2 = bond_ref[:].reshape(MB * L, BF)
    mask = mask_ref[:]
    idx_a = ideg_ref[:]
    idx_b = bdeg_ref[:]

    af2d = _leaky(_mm(atom2, wfT_ref[:]) + bf_ref[:])
    A3 = _mm(atom2, waT_ref[:]).reshape(MB, L, FP)
    B3 = _mm(bond2, wbT_ref[:]).reshape(MB, L, FP)
    bnb = bnb_ref[:][None]
    af3 = af2d.reshape(MB, L, FP)

    iota2 = jax.lax.broadcasted_iota(jnp.int32, (MB, L, 2 * L), 2)

    oh, amask, smask = [], [], []
    for n in range(NBR):
        ia = idx_a[:, :, n]
        ib = idx_b[:, :, n] + L
        oh.append(jnp.where((ia[:, :, None] == iota2) | (ib[:, :, None] == iota2),
                            1.0, 0.0))
        pad = ia == (L - 1)
        amask.append(jnp.where(pad, 0.0, 1.0))
        smask.append(jnp.where(pad, -9.0e8, 0.0))
    OH = jnp.concatenate(oh, axis=1)
    T0 = jnp.concatenate([A3, B3], axis=1)
    N0full = _leaky(_bmm(OH, T0) + bnb)
    N0 = [N0full[:, n * L:(n + 1) * L, :] for n in range(NBR)]
    oh_a = [oh[n][:, :, :L] for n in range(NBR)]

    def attn(s1, s2_list, bal, feats):
        sc = [_leaky(s1 + s2_list[n] + bal) + smask[n] for n in range(NBR)]
        mx = sc[0]
        for n in range(1, NBR):
            mx = jnp.maximum(mx, sc[n])
        e = [jnp.exp(sc[n] - mx) for n in range(NBR)]
        zden = e[0]
        for n in range(1, NBR):
            zden = zden + e[n]
        w = [e[n] / zden * amask[n] for n in range(NBR)]
        wsum = w[0]
        for n in range(1, NBR):
            wsum = wsum + w[n]
        agg = w[0][:, :, None] * feats[0]
        for n in range(1, NBR):
            agg = agg + w[n][:, :, None] * feats[n]
        return agg, wsum, w

    wa1_0 = wa1_0_ref[:][None]
    wa2_0 = wa2_0_ref[:][None]
    s1_0 = jnp.sum(af3 * wa1_0, axis=-1)
    s2_0 = [jnp.sum(N0[n] * wa2_0, axis=-1) for n in range(NBR)]
    agg0, wsum0, _ = attn(s1_0, s2_0, bal0_ref[:], N0)
    ctx0 = _elu(_mm(agg0.reshape(MB * L, FP), wattT0_ref[:]).reshape(MB, L, FP)
                + wsum0[:, :, None] * batt0_ref[:][None])
    h1 = _gru(ctx0.reshape(MB * L, FP), af2d,
              wihT0_ref[:], bih0_ref[:], whhT0_ref[:], bhh0_ref[:])

    act3 = jnp.maximum(h1, 0.0).reshape(MB, L, FP)
    wa1_1 = wa1_1_ref[:][None]
    wa2_1 = wa2_1_ref[:][None]
    s1_1 = jnp.sum(act3 * wa1_1, axis=-1)
    zsc = jnp.sum(act3 * wa2_1, axis=-1)
    s2_1 = [jnp.sum(oh_a[n] * zsc[:, None, :], axis=-1) for n in range(NBR)]
    sc1 = [_leaky(s1_1 + s2_1[n] + bal1_ref[:]) + smask[n] for n in range(NBR)]
    mx = sc1[0]
    for n in range(1, NBR):
        mx = jnp.maximum(mx, sc1[n])
    e1 = [jnp.exp(sc1[n] - mx) for n in range(NBR)]
    zden = e1[0]
    for n in range(1, NBR):
        zden = zden + e1[n]
    w1 = [e1[n] / zden * amask[n] for n in range(NBR)]
    wsum1 = w1[0]
    for n in range(1, NBR):
        wsum1 = wsum1 + w1[n]
    wadj = w1[0][:, :, None] * oh_a[0]
    for n in range(1, NBR):
        wadj = wadj + w1[n][:, :, None] * oh_a[n]
    agg1 = _bmm(wadj, act3)
    ctx1 = _elu(_mm(agg1.reshape(MB * L, FP), wattT1_ref[:]).reshape(MB, L, FP)
                + wsum1[:, :, None] * batt1_ref[:][None])
    h2 = _gru(ctx1.reshape(MB * L, FP), h1,
              wihT1_ref[:], bih1_ref[:], whhT1_ref[:], bhh1_ref[:])

    act5 = jnp.maximum(h2, 0.0).reshape(MB, L, FP)
    mol = jnp.sum(act5 * mask[:, :, None], axis=1)
    act_mol = jnp.maximum(mol, 0.0)
    mmask_add = jnp.where(mask == 0.0, -9.0e8, 0.0)
    wm1 = wm1_ref[:]
    wm2 = wm2_ref[:][None]
    for _t in range(2):
        s1m = jnp.sum(act_mol * wm1, axis=-1, keepdims=True)
        s2m = jnp.sum(act5 * wm2, axis=-1)
        sc = _leaky(s1m + s2m + bm_ref[:]) + mmask_add
        mx2 = jnp.max(sc, axis=1, keepdims=True)
        e2 = jnp.exp(sc - mx2)
        w2 = e2 / jnp.sum(e2, axis=1, keepdims=True) * mask
        wsum2 = jnp.sum(w2, axis=1, keepdims=True)
        aggm = jnp.sum(act5 * w2[:, :, None], axis=1)
        ctxm = _elu(_mm(aggm, wmaT_ref[:]) + wsum2 * bma_ref[:])
        mol_new = _gru(ctxm, mol, wihTm_ref[:], bihm_ref[:],
                       whhTm_ref[:], bhhm_ref[:])
        mu = jnp.mean(mol_new, axis=1, keepdims=True)
        var = jnp.mean((mol_new - mu) ** 2, axis=1, keepdims=True)
        mol = (mol_new - mu) * jax.lax.rsqrt(var + 1e-5) * lng_ref[:] \
            + lnb_ref[:] + mol_new
        act_mol = jnp.maximum(mol, 0.0)
    out_ref[:] = mol


def _xyz_body(xyz_ref, mol_ref,
              w1T_ref, g1_ref, be1_ref, w2T_ref, g2_ref, be2_ref,
              w3T_ref, g3_ref, be3_ref,
              fc1T_ref, bfc1_ref, fc2T_ref, bfc2_ref,
              fc3T_ref, bfc3_ref, fc4T_ref, bfc4_ref, out_ref):
    N = B * L
    x0 = xyz_ref[:]
    z1 = _mm(x0, w1T_ref[:])
    mu1 = jnp.mean(z1, axis=0, keepdims=True)
    v1 = jnp.mean(z1 * z1, axis=0, keepdims=True) - mu1 * mu1
    x1 = _leaky((z1 - mu1) * jax.lax.rsqrt(v1 + 1e-5) * g1_ref[:] + be1_ref[:])
    z2 = _mm(x1, w2T_ref[:])
    mu2 = jnp.mean(z2, axis=0, keepdims=True)
    v2 = jnp.mean(z2 * z2, axis=0, keepdims=True) - mu2 * mu2
    x2 = _leaky((z2 - mu2) * jax.lax.rsqrt(v2 + 1e-5) * g2_ref[:] + be2_ref[:])

    CH = 2048
    ssum = jnp.zeros((1, 1024), jnp.float32)
    ssq = jnp.zeros((1, 1024), jnp.float32)
    mxs, mns = [], []
    for c in range(N // CH):
        z3c = _mm(x2[c * CH:(c + 1) * CH], w3T_ref[:])
        ssum = ssum + jnp.sum(z3c, axis=0, keepdims=True)
        ssq = ssq + jnp.sum(z3c * z3c, axis=0, keepdims=True)
        z3r = z3c.reshape(CH // L, L, 1024)
        mxs.append(jnp.max(z3r, axis=1))
        mns.append(jnp.min(z3r, axis=1))
    maxb = jnp.concatenate(mxs, axis=0)
    minb = jnp.concatenate(mns, axis=0)
    mu3 = ssum / N
    v3 = ssq / N - mu3 * mu3
    scale = jax.lax.rsqrt(v3 + 1e-5) * g3_ref[:]
    f_mx = _leaky((maxb - mu3) * scale + be3_ref[:])
    f_mn = _leaky((minb - mu3) * scale + be3_ref[:])
    x3 = jnp.maximum(f_mx, f_mn)
    h = jnp.maximum(_mm(x3, fc1T_ref[:]) + bfc1_ref[:], 0.0)
    h = jnp.maximum(_mm(h, fc2T_ref[:]) + bfc2_ref[:], 0.0)
    cat = jnp.concatenate([mol_ref[:], h], axis=1)
    f3 = jnp.maximum(_mm(cat, fc3T_ref[:]) + bfc3_ref[:], 0.0)
    logits = _mm(f3, fc4T_ref[:]) + bfc4_ref[:]
    mxl = jnp.max(logits, axis=1, keepdims=True)
    el = jnp.exp(logits - mxl)
    out_ref[:] = el / jnp.sum(el, axis=1, keepdims=True)


def _row(v):
    return jnp.asarray(v, jnp.float32).reshape(1, -1)


def kernel(params, atom_list, bond_list, atom_degree_list, bond_degree_list,
           atom_mask, xyz_feature):
    p = params
    wfT = p['atom_fc'][0].T
    bf = _row(p['atom_fc'][1])
    waT = p['neighbor_fc'][0][:, :AF].T
    wbT = p['neighbor_fc'][0][:, AF:].T
    bnb = _row(p['neighbor_fc'][1])

    def align_parts(r):
        W, b = p['align'][r]
        return _row(W[0, :FP]), _row(W[0, FP:]), _row(b)

    wa1_0, wa2_0, bal0 = align_parts(0)
    wa1_1, wa2_1, bal1 = align_parts(1)
    wattT0, batt0 = p['attend'][0][0].T, _row(p['attend'][0][1])
    wattT1, batt1 = p['attend'][1][0].T, _row(p['attend'][1][1])

    def gru_parts(g):
        return (g['W_ih'].T, _row(g['b_ih']), g['W_hh'].T, _row(g['b_hh']))

    wihT0, bih0, whhT0, bhh0 = gru_parts(p['gru'][0])
    wihT1, bih1, whhT1, bhh1 = gru_parts(p['gru'][1])
    wihTm, bihm, whhTm, bhhm = gru_parts(p['mol_gru'])
    Wm, bm = p['mol_align']
    wm1, wm2, bmr = _row(Wm[0, :FP]), _row(Wm[0, FP:]), _row(bm)
    wmaT, bma = p['mol_attend'][0].T, _row(p['mol_attend'][1])
    lng, lnb = _row(p['ln'][0]), _row(p['ln'][1])

    full = lambda shape: pl.BlockSpec(shape, lambda i: (0,) * len(shape))
    weight_args = [wfT, bf, waT, wbT, bnb,
                   wa1_0, wa2_0, bal0, wattT0, batt0, wihT0, bih0, whhT0, bhh0,
                   wa1_1, wa2_1, bal1, wattT1, batt1, wihT1, bih1, whhT1, bhh1,
                   wm1, wm2, bmr, wmaT, bma, wihTm, bihm, whhTm, bhhm, lng, lnb]

    mol_feature = pl.pallas_call(
        _gnn_body,
        grid=(B // MB,),
        in_specs=[
            pl.BlockSpec((MB, L, AF), lambda i: (i, 0, 0)),
            pl.BlockSpec((MB, L, BF), lambda i: (i, 0, 0)),
            pl.BlockSpec((MB, L, NBR), lambda i: (i, 0, 0)),
            pl.BlockSpec((MB, L, NBR), lambda i: (i, 0, 0)),
            pl.BlockSpec((MB, L), lambda i: (i, 0)),
        ] + [full(w.shape) for w in weight_args],
        out_specs=pl.BlockSpec((MB, FP), lambda i: (i, 0)),
        out_shape=jax.ShapeDtypeStruct((B, FP), jnp.float32),
    )(atom_list, bond_list, atom_degree_list, bond_degree_list, atom_mask,
      *weight_args)

    xyz2 = xyz_feature.reshape(B * L, 6)
    head_args = [p['conv1'][0].T, _row(p['bn1'][0]), _row(p['bn1'][1]),
                 p['conv2'][0].T, _row(p['bn2'][0]), _row(p['bn2'][1]),
                 p['conv3'][0].T, _row(p['bn3'][0]), _row(p['bn3'][1]),
                 p['fc1'][0].T, _row(p['fc1'][1]),
                 p['fc2'][0].T, _row(p['fc2'][1]),
                 p['fc3'][0].T, _row(p['fc3'][1]),
                 p['fc4'][0].T, _row(p['fc4'][1])]

    out = pl.pallas_call(
        _xyz_body,
        out_shape=jax.ShapeDtypeStruct((B, NC), jnp.float32),
    )(xyz2, mol_feature, *head_args)
    return out
